# baseline (speedup 1.0000x reference)
import numpy as np

import jax
import jax.numpy as jnp
from jax import lax
from jax.experimental import pallas as pl
from jax.experimental.pallas import tpu as pltpu

B, N = 128, 100000
RB = 8
CB = 8192
NR = B // RB
NC = (N + CB - 1) // CB


def _gumbel_const():
    n = B * N
    ks = (np.uint32(0), np.uint32(1),
          np.uint32(0) ^ np.uint32(1) ^ np.uint32(0x1BD11BDA))
    rots = (np.array([13, 15, 26, 6], np.uint32),
            np.array([17, 29, 16, 24], np.uint32))
    with np.errstate(over="ignore"):
        x1 = np.arange(n, dtype=np.uint32)
        x0 = np.full(n, ks[0], dtype=np.uint32)
        x1 += ks[1]
        for grp in range(5):
            for r in rots[grp % 2]:
                x0 += x1
                x1 = ((x1 << r) | (x1 >> np.uint32(32 - r)))
                x1 ^= x0
            x0 += ks[(grp + 1) % 3]
            x1 += ks[(grp + 2) % 3] + np.uint32(grp + 1)
        bits = x0 ^ x1
    fb = (bits >> np.uint32(9)) | np.uint32(0x3F800000)
    u = fb.view(np.float32) - np.float32(1.0)
    u = np.maximum(u, np.float32(np.finfo(np.float32).tiny))
    g = -np.log(-np.log(u, dtype=np.float32), dtype=np.float32)
    return g.reshape(B, N)


_GT = np.ascontiguousarray(_gumbel_const().T)


CBT = 10000
NCT = N // CBT


MCBT = 2000
MNCT = N // MCBT
XPRE = 6


def _manual_kernel(x_hbm, g_hbm, p_hbm, a_ref,
                   xbuf, gbuf, pbuf, m_ref, l_ref, bv_ref,
                   xsem, gsem, psem):
    t = pl.program_id(0)

    def x_copy(j):
        return pltpu.make_async_copy(
            x_hbm.at[pl.ds(j * MCBT, MCBT), :],
            xbuf.at[pl.ds(j * MCBT, MCBT), :],
            xsem.at[j])

    def g_copy(j, slot):
        return pltpu.make_async_copy(
            g_hbm.at[pl.ds(j * MCBT, MCBT), :], gbuf.at[slot], gsem.at[slot])

    def p_copy(j, slot):
        return pltpu.make_async_copy(
            pbuf.at[slot], p_hbm.at[pl.ds(j * MCBT, MCBT), :], psem.at[slot])

    @pl.when(t == 0)
    def _prologue():
        for jj in range(XPRE):
            x_copy(jj).start()

    @pl.when(t < MNCT)
    def _phase_stats():
        j = t
        x_copy(j).wait()

        x = xbuf[pl.ds(j * MCBT, MCBT), :]
        pm = jnp.max(x, axis=0, keepdims=True)

        @pl.when(j == 0)
        def _():
            m_ref[...] = pm
            l_ref[...] = jnp.sum(jnp.exp(x - pm), axis=0, keepdims=True)

        @pl.when(j > 0)
        def _():
            m_old = m_ref[...]
            m_new = jnp.maximum(m_old, pm)
            l_ref[...] = (l_ref[...] * jnp.exp(m_old - m_new)
                          + jnp.sum(jnp.exp(x - m_new), axis=0,
                                    keepdims=True))
            m_ref[...] = m_new

        @pl.when(j + XPRE < MNCT)
        def _():
            x_copy(j + XPRE).start()

        @pl.when(j == MNCT - 1)
        def _():
            g_copy(0, 0).start()
            g_copy(1, 1).start()

    @pl.when(t >= MNCT)
    def _phase_probs():
        jb = t - MNCT
        slot = lax.rem(jb, 3)

        @pl.when(jb >= 3)
        def _():
            p_copy(jb - 3, slot).wait()

        gslot = lax.rem(jb, 2)
        g_copy(jb, gslot).wait()

        x = xbuf[pl.ds(jb * MCBT, MCBT), :]
        v = x + gbuf[gslot]
        pv = jnp.max(v, axis=0, keepdims=True)
        rid = lax.broadcasted_iota(jnp.int32, (MCBT, B), 0) + jb * MCBT
        pidx = jnp.min(jnp.where(v == pv, rid, jnp.int32(2**30)),
                       axis=0, keepdims=True)

        @pl.when(jb == 0)
        def _():
            bv_ref[...] = pv
            a_ref[...] = pidx

        @pl.when(jb > 0)
        def _():
            bv = bv_ref[...]
            better = pv > bv
            bv_ref[...] = jnp.where(better, pv, bv)
            a_ref[...] = jnp.where(better, pidx, a_ref[...])

        pbuf[slot] = jnp.exp(x - m_ref[...]) * (1.0 / l_ref[...])
        p_copy(jb, slot).start()

        @pl.when(jb + 2 < MNCT)
        def _():
            g_copy(jb + 2, gslot).start()

        @pl.when(t == 2 * MNCT - 1)
        def _epilogue():
            p_copy(jb - 2, lax.rem(jb + 1, 3)).wait()
            p_copy(jb - 1, lax.rem(jb + 2, 3)).wait()
            p_copy(jb, slot).wait()


def _stats_kernel(x_ref, g_ref, m_ref, l_ref, a_ref, bv_ref):
    j = pl.program_id(0)
    x = x_ref[...]
    v = x + g_ref[...]

    pm = jnp.max(x, axis=0, keepdims=True)
    pv = jnp.max(v, axis=0, keepdims=True)
    rid = lax.broadcasted_iota(jnp.int32, (CBT, B), 0) + j * CBT
    pidx = jnp.min(jnp.where(v == pv, rid, jnp.int32(2**30)),
                   axis=0, keepdims=True)

    @pl.when(j == 0)
    def _():
        m_ref[...] = pm
        l_ref[...] = jnp.sum(jnp.exp(x - pm), axis=0, keepdims=True)
        bv_ref[...] = pv
        a_ref[...] = pidx

    @pl.when(j > 0)
    def _():
        m_old = m_ref[...]
        m_new = jnp.maximum(m_old, pm)
        l_ref[...] = (l_ref[...] * jnp.exp(m_old - m_new)
                      + jnp.sum(jnp.exp(x - m_new), axis=0, keepdims=True))
        m_ref[...] = m_new
        bv = bv_ref[...]
        better = pv > bv
        bv_ref[...] = jnp.where(better, pv, bv)
        a_ref[...] = jnp.where(better, pidx, a_ref[...])


def _probs_kernel(x_ref, m_ref, l_ref, p_ref):
    p_ref[...] = jnp.exp(x_ref[...] - m_ref[...]) / l_ref[...]


def kernel(s):
    x = s.T

    probs_t, a = pl.pallas_call(
        _manual_kernel,
        grid=(2 * MNCT,),
        in_specs=[
            pl.BlockSpec(memory_space=pltpu.MemorySpace.HBM),
            pl.BlockSpec(memory_space=pltpu.MemorySpace.HBM),
        ],
        out_specs=[
            pl.BlockSpec(memory_space=pltpu.MemorySpace.HBM),
            pl.BlockSpec((1, B), lambda t: (0, 0)),
        ],
        out_shape=[
            jax.ShapeDtypeStruct((N, B), jnp.float32),
            jax.ShapeDtypeStruct((1, B), jnp.int32),
        ],
        scratch_shapes=[
            pltpu.VMEM((N, B), jnp.float32),
            pltpu.VMEM((2, MCBT, B), jnp.float32),
            pltpu.VMEM((3, MCBT, B), jnp.float32),
            pltpu.VMEM((1, B), jnp.float32),
            pltpu.VMEM((1, B), jnp.float32),
            pltpu.VMEM((1, B), jnp.float32),
            pltpu.SemaphoreType.DMA((MNCT,)),
            pltpu.SemaphoreType.DMA((2,)),
            pltpu.SemaphoreType.DMA((3,)),
        ],
        compiler_params=pltpu.CompilerParams(
            dimension_semantics=("arbitrary",)),
    )(x, _GT)

    return probs_t.T, a.reshape(B)


def _kernel_twopass(s):
    x = s.T

    m, l, a = pl.pallas_call(
        _stats_kernel,
        grid=(NCT,),
        in_specs=[
            pl.BlockSpec((CBT, B), lambda j: (j, 0)),
            pl.BlockSpec((CBT, B), lambda j: (j, 0)),
        ],
        out_specs=[
            pl.BlockSpec((1, B), lambda j: (0, 0)),
            pl.BlockSpec((1, B), lambda j: (0, 0)),
            pl.BlockSpec((1, B), lambda j: (0, 0)),
        ],
        out_shape=[
            jax.ShapeDtypeStruct((1, B), jnp.float32),
            jax.ShapeDtypeStruct((1, B), jnp.float32),
            jax.ShapeDtypeStruct((1, B), jnp.int32),
        ],
        scratch_shapes=[pltpu.VMEM((1, B), jnp.float32)],
        compiler_params=pltpu.CompilerParams(
            dimension_semantics=("arbitrary",)),
    )(x, _GT)

    probs_t = pl.pallas_call(
        _probs_kernel,
        grid=(NCT,),
        in_specs=[
            pl.BlockSpec((CBT, B), lambda j: (j, 0)),
            pl.BlockSpec((1, B), lambda j: (0, 0)),
            pl.BlockSpec((1, B), lambda j: (0, 0)),
        ],
        out_specs=pl.BlockSpec((CBT, B), lambda j: (j, 0)),
        out_shape=jax.ShapeDtypeStruct((N, B), jnp.float32),
        compiler_params=pltpu.CompilerParams(
            dimension_semantics=("parallel",)),
    )(x, m, l)

    return probs_t.T, a.reshape(B)


# speedup vs baseline: 1.0678x; 1.0678x over previous
import numpy as np

import jax
import jax.numpy as jnp
from jax import lax
from jax.experimental import pallas as pl
from jax.experimental.pallas import tpu as pltpu

B, N = 128, 100000
RB = 8
CB = 8192
NR = B // RB
NC = (N + CB - 1) // CB


def _gumbel_const():
    n = B * N
    ks = (np.uint32(0), np.uint32(1),
          np.uint32(0) ^ np.uint32(1) ^ np.uint32(0x1BD11BDA))
    rots = (np.array([13, 15, 26, 6], np.uint32),
            np.array([17, 29, 16, 24], np.uint32))
    with np.errstate(over="ignore"):
        x1 = np.arange(n, dtype=np.uint32)
        x0 = np.full(n, ks[0], dtype=np.uint32)
        x1 += ks[1]
        for grp in range(5):
            for r in rots[grp % 2]:
                x0 += x1
                x1 = ((x1 << r) | (x1 >> np.uint32(32 - r)))
                x1 ^= x0
            x0 += ks[(grp + 1) % 3]
            x1 += ks[(grp + 2) % 3] + np.uint32(grp + 1)
        bits = x0 ^ x1
    fb = (bits >> np.uint32(9)) | np.uint32(0x3F800000)
    u = fb.view(np.float32) - np.float32(1.0)
    u = np.maximum(u, np.float32(np.finfo(np.float32).tiny))
    g = -np.log(-np.log(u, dtype=np.float32), dtype=np.float32)
    return g.reshape(B, N)


_GT = np.ascontiguousarray(_gumbel_const().T)


CBT = 10000
NCT = N // CBT


def _stats_kernel(x_ref, g_ref, m_ref, l_ref, a_ref, bv_ref):
    j = pl.program_id(0)
    x = x_ref[...]
    v = x + g_ref[...]

    pm = jnp.max(x, axis=0, keepdims=True)
    pv = jnp.max(v, axis=0, keepdims=True)
    rid = lax.broadcasted_iota(jnp.int32, (CBT, B), 0) + j * CBT
    pidx = jnp.min(jnp.where(v == pv, rid, jnp.int32(2**30)),
                   axis=0, keepdims=True)

    @pl.when(j == 0)
    def _():
        m_ref[...] = pm
        l_ref[...] = jnp.sum(jnp.exp(x - pm), axis=0, keepdims=True)
        bv_ref[...] = pv
        a_ref[...] = pidx

    @pl.when(j > 0)
    def _():
        m_old = m_ref[...]
        m_new = jnp.maximum(m_old, pm)
        l_ref[...] = (l_ref[...] * jnp.exp(m_old - m_new)
                      + jnp.sum(jnp.exp(x - m_new), axis=0, keepdims=True))
        m_ref[...] = m_new
        bv = bv_ref[...]
        better = pv > bv
        bv_ref[...] = jnp.where(better, pv, bv)
        a_ref[...] = jnp.where(better, pidx, a_ref[...])


def _probs_kernel(x_ref, m_ref, l_ref, p_ref):
    p_ref[...] = jnp.exp(x_ref[...] - m_ref[...]) / l_ref[...]


def kernel(s):
    x = s.T

    m, l, a = pl.pallas_call(
        _stats_kernel,
        grid=(NCT,),
        in_specs=[
            pl.BlockSpec((CBT, B), lambda j: (j, 0)),
            pl.BlockSpec((CBT, B), lambda j: (j, 0)),
        ],
        out_specs=[
            pl.BlockSpec((1, B), lambda j: (0, 0)),
            pl.BlockSpec((1, B), lambda j: (0, 0)),
            pl.BlockSpec((1, B), lambda j: (0, 0)),
        ],
        out_shape=[
            jax.ShapeDtypeStruct((1, B), jnp.float32),
            jax.ShapeDtypeStruct((1, B), jnp.float32),
            jax.ShapeDtypeStruct((1, B), jnp.int32),
        ],
        scratch_shapes=[pltpu.VMEM((1, B), jnp.float32)],
        compiler_params=pltpu.CompilerParams(
            dimension_semantics=("arbitrary",)),
    )(x, _GT)

    probs_t = pl.pallas_call(
        _probs_kernel,
        grid=(NCT,),
        in_specs=[
            pl.BlockSpec((CBT, B), lambda j: (j, 0)),
            pl.BlockSpec((1, B), lambda j: (0, 0)),
            pl.BlockSpec((1, B), lambda j: (0, 0)),
        ],
        out_specs=pl.BlockSpec((CBT, B), lambda j: (j, 0)),
        out_shape=jax.ShapeDtypeStruct((N, B), jnp.float32),
        compiler_params=pltpu.CompilerParams(
            dimension_semantics=("parallel",)),
    )(x, m, l)

    return probs_t.T, a.reshape(B)
